# Initial kernel scaffold; baseline (speedup 1.0000x reference)
import jax
import jax.numpy as jnp
from jax.experimental import pallas as pl


def kernel(x, gw1, gb1, gw2, gb2, We1, be1, ln_g, ln_b, We2, be2):
    raise NotImplementedError("write your pallas kernel here")


# speedup vs baseline: 3.5797x; 3.5797x over previous
import jax
import jax.numpy as jnp
from jax import lax
from jax.experimental import pallas as pl
from jax.experimental.pallas import tpu as pltpu

_B = 8192
_D = 768
_H = 256
_C = 2
_E = 8
_GH = 128
_BT = 512


def _moe_body(x_ref, gw1_ref, gb1_ref, gw2_ref, gb2_ref,
              We1_ref, be1_ref, ln_g_ref, ln_b_ref, We2_ref, be2_ref,
              scores_ref, idx_ref, out8_ref,
              s_acc, wsum_acc):
    step = pl.program_id(0)
    nsteps = pl.num_programs(0)
    xb = x_ref[...]

    g1 = jnp.dot(xb, gw1_ref[...], preferred_element_type=jnp.float32)
    g1 = jnp.maximum(g1 + gb1_ref[...], 0.0)
    logits = jnp.dot(g1, gw2_ref[...], preferred_element_type=jnp.float32)
    logits = logits + gb2_ref[...]
    mx = jnp.max(logits, axis=-1, keepdims=True)
    ex = jnp.exp(logits - mx)
    scores = ex / jnp.sum(ex, axis=-1, keepdims=True)
    scores_ref[...] = scores

    eiota = lax.broadcasted_iota(jnp.int32, (_BT, _E), 1)
    m1 = jnp.max(scores, axis=-1, keepdims=True)
    i1 = jnp.min(jnp.where(scores == m1, eiota, _E), axis=-1, keepdims=True)
    masked = jnp.where(eiota == i1, -1.0, scores)
    m2 = jnp.max(masked, axis=-1, keepdims=True)
    i2 = jnp.min(jnp.where(masked == m2, eiota, _E), axis=-1, keepdims=True)
    idx_ref[...] = jnp.concatenate([i1, i2], axis=1)
    denom = m1 + m2
    w1 = m1 / denom
    w2 = m2 / denom
    gates = jnp.where(eiota == i1, w1, 0.0) + jnp.where(eiota == i2, w2, 0.0)

    @pl.when(step == 0)
    def _init():
        s_acc[...] = jnp.zeros_like(s_acc)
        for e in range(_E):
            wsum_acc[0, e] = 0.0

    for e in range(_E):
        h = jnp.dot(xb, We1_ref[e], preferred_element_type=jnp.float32)
        h = h + be1_ref[e:e + 1, :]
        h = 0.5 * h * (1.0 + lax.erf(h * 0.70710678118654752))
        mu = jnp.mean(h, axis=-1, keepdims=True)
        cen = h - mu
        var = jnp.mean(cen * cen, axis=-1, keepdims=True)
        hn = cen / jnp.sqrt(var + 1e-5)
        hn = hn * ln_g_ref[e:e + 1, :] + ln_b_ref[e:e + 1, :]
        ge = gates[:, e:e + 1]
        s_acc[e:e + 1, :] += jnp.sum(hn * ge, axis=0, keepdims=True)
        wsum_acc[0, e] += jnp.sum(ge)

    @pl.when(step == nsteps - 1)
    def _finish():
        for e in range(_E):
            o = jnp.dot(s_acc[e:e + 1, :], We2_ref[e],
                        preferred_element_type=jnp.float32)
            out8_ref[e:e + 1, :] = o + be2_ref[e:e + 1, :] * wsum_acc[0, e]


def kernel(x, gw1, gb1, gw2, gb2, We1, be1, ln_g, ln_b, We2, be2):
    nsteps = _B // _BT
    full = lambda i: (0, 0)
    full3 = lambda i: (0, 0, 0)
    scores, idx, out8 = pl.pallas_call(
        _moe_body,
        grid=(nsteps,),
        in_specs=[
            pl.BlockSpec((_BT, _D), lambda i: (i, 0)),
            pl.BlockSpec((_D, _GH), full),
            pl.BlockSpec((1, _GH), full),
            pl.BlockSpec((_GH, _E), full),
            pl.BlockSpec((1, _E), full),
            pl.BlockSpec((_E, _D, _H), full3),
            pl.BlockSpec((_E, _H), full),
            pl.BlockSpec((_E, _H), full),
            pl.BlockSpec((_E, _H), full),
            pl.BlockSpec((_E, _H, _C), full3),
            pl.BlockSpec((_E, _C), full),
        ],
        out_specs=[
            pl.BlockSpec((_BT, _E), lambda i: (i, 0)),
            pl.BlockSpec((_BT, 2), lambda i: (i, 0)),
            pl.BlockSpec((_E, _C), full),
        ],
        out_shape=[
            jax.ShapeDtypeStruct((_B, _E), jnp.float32),
            jax.ShapeDtypeStruct((_B, 2), jnp.int32),
            jax.ShapeDtypeStruct((_E, _C), jnp.float32),
        ],
        scratch_shapes=[
            pltpu.VMEM((_E, _H), jnp.float32),
            pltpu.SMEM((1, _E), jnp.float32),
        ],
    )(x, gw1, gb1.reshape(1, _GH), gw2, gb2.reshape(1, _E),
      We1, be1, ln_g, ln_b, We2, be2)
    output = jnp.zeros((_B, _C), jnp.float32).at[:_E, :].set(out8)
    return output, scores, idx
